# baseline (speedup 1.0000x reference)
import functools

import jax
import jax.numpy as jnp
from jax import lax
from jax.experimental import pallas as pl
from jax.experimental.pallas import tpu as pltpu
from jax.experimental.pallas import tpu_sc as plsc

_B = 16384
_D = 64
_NPAIR = 5
_NW = 32
_PER_W = _B // _NW
_C = 128
_NCHUNK = _PER_W // _C
_G = _C // 16


def _sc_predictions_kernel(user_table, item_table, uid_hbm, iidT_hbm, out_hbm,
                           uids_v, iids_v, u_slab, i_slab, pred_v, sem):
    wid = lax.axis_index("s") * 2 + lax.axis_index("c")
    lane = lax.iota(jnp.int32, 16)

    for c in range(_NCHUNK):
        base = wid * _PER_W + c * _C
        pltpu.sync_copy(uid_hbm.at[pl.ds(base, _C)], uids_v)
        for j in range(_NPAIR):
            pltpu.sync_copy(iidT_hbm.at[pl.ds(j * _B + base, _C)],
                            iids_v.at[pl.ds(j * _C, _C)])
        cps = [pltpu.async_copy(user_table.at[uids_v], u_slab, sem)]
        for j in range(_NPAIR):
            cps.append(pltpu.async_copy(
                item_table.at[iids_v.at[pl.ds(j * _C, _C)]],
                i_slab.at[pl.ds(j * _C, _C)], sem))
        for cp in cps:
            cp.wait()

        def group_body(g, carry):
            b0 = g * 16
            res = [jnp.zeros((16,), jnp.float32) for _ in range(_NPAIR)]
            for p in range(16):
                b = b0 + p
                us = [u_slab[b, pl.ds(q * 16, 16)] for q in range(_D // 16)]
                for j in range(_NPAIR):
                    r = j * _C + b
                    prod = us[0] * i_slab[r, pl.ds(0, 16)]
                    for q in range(1, _D // 16):
                        prod = prod + us[q] * i_slab[r, pl.ds(q * 16, 16)]
                    s = jnp.sum(prod)
                    res[j] = jnp.where(lane == p, s, res[j])
            for j in range(_NPAIR):
                pred_v[pl.ds(j * _C + b0, 16)] = res[j]
            return carry

        lax.fori_loop(0, _G, group_body, 0)
        pltpu.sync_copy(
            pred_v,
            out_hbm.at[pl.ds((wid * _NCHUNK + c) * _NPAIR * _C, _NPAIR * _C)])


def _tc_loss_kernel(pred_ref, out_ref):
    p = pred_ref[...]
    pos = p[:, 0:1, :]
    negs = p[:, 1:_NPAIR, :]
    out_ref[...] = jnp.mean(jax.nn.softplus(negs - pos)).reshape(1, 1)


def kernel(user_table, item_table, user_id, item_id):
    uid = user_id.reshape(_B)
    iidT = item_id.T.reshape(_NPAIR * _B)

    sc = pl.kernel(
        _sc_predictions_kernel,
        out_type=jax.ShapeDtypeStruct((_NW * _NCHUNK * _NPAIR * _C,),
                                      jnp.float32),
        mesh=plsc.VectorSubcoreMesh(core_axis_name="c", subcore_axis_name="s"),
        compiler_params=pltpu.CompilerParams(
            needs_layout_passes=False, use_tc_tiling_on_sc=False),
        scratch_types=[
            pltpu.VMEM((_C,), jnp.int32),
            pltpu.VMEM((_NPAIR * _C,), jnp.int32),
            pltpu.VMEM((_C, _D), jnp.float32),
            pltpu.VMEM((_NPAIR * _C, _D), jnp.float32),
            pltpu.VMEM((_NPAIR * _C,), jnp.float32),
            pltpu.SemaphoreType.DMA,
        ],
    )
    preds = sc(user_table, item_table, uid, iidT)
    preds = preds.reshape(_NW * _NCHUNK, _NPAIR, _C)

    loss = pl.pallas_call(
        _tc_loss_kernel,
        out_shape=jax.ShapeDtypeStruct((1, 1), jnp.float32),
    )(preds)
    return loss[0, 0]
